# speedup vs baseline: 2.3387x; 2.3387x over previous
import functools

import jax
import jax.numpy as jnp
from jax import lax
from jax.experimental import pallas as pl
from jax.experimental.pallas import tpu as pltpu
from jax.experimental.pallas import tpu_sc as plsc

B = 16384
H = 128
NC = 2
NS = 16
NW = NC * NS
BPW = B // NW
CHUNK = 128
NCHUNK = BPW // CHUNK

BM = 2048


def _gather_body(pt_hbm, ct_hbm, bt_hbm, ip_hbm, ic_hbm, ib_hbm,
                 outp_hbm, outc_hbm, outb_hbm,
                 idx_v, buf_v, sem):
    wid = lax.axis_index("s") * NC + lax.axis_index("c")
    base_row = wid * NCHUNK
    base = wid * BPW
    for tab, idx_hbm, out_hbm in (
        (pt_hbm, ip_hbm, outp_hbm),
        (ct_hbm, ic_hbm, outc_hbm),
        (bt_hbm, ib_hbm, outb_hbm),
    ):
        pltpu.sync_copy(idx_hbm.at[pl.ds(base_row, NCHUNK)], idx_v)
        descs = [
            pltpu.async_copy(tab.at[idx_v.at[j]],
                             buf_v.at[pl.ds(j * CHUNK, CHUNK)], sem)
            for j in range(NCHUNK)
        ]
        for d in descs:
            d.wait()
        pltpu.sync_copy(buf_v, out_hbm.at[pl.ds(base, BPW)])


_gather = pl.kernel(
    _gather_body,
    out_type=(
        jax.ShapeDtypeStruct((B, H), jnp.float32),
        jax.ShapeDtypeStruct((B, H), jnp.float32),
        jax.ShapeDtypeStruct((B, H), jnp.float32),
    ),
    mesh=plsc.VectorSubcoreMesh(core_axis_name="c", subcore_axis_name="s",
                                num_cores=NC, num_subcores=NS),
    scratch_types=[
        pltpu.VMEM((NCHUNK, CHUNK), jnp.int32),
        pltpu.VMEM((BPW, H), jnp.float32),
        pltpu.SemaphoreType.DMA,
    ],
)


def _mlp_body(gp_ref, gc_ref, gb_ref, w1_ref, b1_ref, w2_ref, b2_ref, out_ref):
    h = gp_ref[...] + gc_ref[...] + gb_ref[...]
    a = jnp.dot(h, w1_ref[...], preferred_element_type=jnp.float32) + b1_ref[...]
    a = a * jax.nn.sigmoid(a)
    out_ref[...] = (jnp.dot(a, w2_ref[...], preferred_element_type=jnp.float32)
                    + b2_ref[...])


def _mlp(gp, gc, gb, W1, b1, W2, b2):
    grid = (B // BM,)
    row_spec = pl.BlockSpec((BM, H), lambda i: (i, 0))
    full = pl.BlockSpec((H, H), lambda i: (0, 0))
    bias = pl.BlockSpec((1, H), lambda i: (0, 0))
    return pl.pallas_call(
        _mlp_body,
        grid=grid,
        in_specs=[row_spec, row_spec, row_spec, full, bias, full, bias],
        out_specs=row_spec,
        out_shape=jax.ShapeDtypeStruct((B, H), jnp.float32),
    )(gp, gc, gb, W1, b1.reshape(1, H), W2, b2.reshape(1, H))


def kernel(perturbation, cell_type, batch, perturb_table, cell_table,
           batch_table, W1, b1, W2, b2):
    ip = perturbation.astype(jnp.int32).reshape(B // CHUNK, CHUNK)
    ic = cell_type.astype(jnp.int32).reshape(B // CHUNK, CHUNK)
    ib = batch.astype(jnp.int32).reshape(B // CHUNK, CHUNK)
    gp, gc, gb = _gather(perturb_table, cell_table, batch_table, ip, ic, ib)
    return _mlp(gp, gc, gb, W1, b1, W2, b2)


# speedup vs baseline: 2.8287x; 1.2095x over previous
import functools

import jax
import jax.numpy as jnp
from jax import lax
from jax.experimental import pallas as pl
from jax.experimental.pallas import tpu as pltpu
from jax.experimental.pallas import tpu_sc as plsc

B = 16384
H = 128
NC = 2
NS = 16
NW = NC * NS
BPW = B // NW
CHUNK = 128
NCHUNK = BPW // CHUNK
NVEC = H // 16

BM = 2048


def _gather_body(pt_hbm, ct_hbm, bt_hbm, ip_hbm, ic_hbm, ib_hbm,
                 out_hbm,
                 idxp, idxc, idxb, acc, buf, isem, psem, bsem0, bsem1):
    wid = lax.axis_index("s") * NC + lax.axis_index("c")
    base_row = wid * NCHUNK
    base = wid * BPW

    idescs = [
        pltpu.async_copy(ip_hbm.at[pl.ds(base_row, NCHUNK)], idxp, isem),
        pltpu.async_copy(ic_hbm.at[pl.ds(base_row, NCHUNK)], idxc, isem),
        pltpu.async_copy(ib_hbm.at[pl.ds(base_row, NCHUNK)], idxb, isem),
    ]
    for d in idescs:
        d.wait()

    pdescs = [
        pltpu.async_copy(pt_hbm.at[idxp.at[j]],
                         acc.at[pl.ds(j * CHUNK, CHUNK)], psem)
        for j in range(NCHUNK)
    ]

    segs = ([(ct_hbm, idxc, j) for j in range(NCHUNK)]
            + [(bt_hbm, idxb, j) for j in range(NCHUNK)])
    bsems = (bsem0, bsem1)

    def fire(s):
        tab, idx, j = segs[s]
        k = s % 2
        return pltpu.async_copy(tab.at[idx.at[j]], buf.at[k], bsems[k])

    descs = {0: fire(0)}
    for d in pdescs:
        d.wait()

    for s in range(len(segs)):
        k = s % 2
        if s + 1 < len(segs):
            descs[s + 1] = fire(s + 1)
        descs.pop(s).wait()
        cbase = segs[s][2] * CHUNK

        def add_body(i, _, k=k, cbase=cbase):
            r0 = i * 2
            r1 = r0 + 1
            for u in range(NVEC):
                c = u * 16
                plsc.addupdate(acc.at[cbase + r0, pl.ds(c, 16)],
                               buf[k, r0, pl.ds(c, 16)])
                plsc.addupdate(acc.at[cbase + r1, pl.ds(c, 16)],
                               buf[k, r1, pl.ds(c, 16)])
            return _

        lax.fori_loop(0, CHUNK // 2, add_body, None)

    pltpu.sync_copy(acc, out_hbm.at[pl.ds(base, BPW)])


_gather = pl.kernel(
    _gather_body,
    out_type=jax.ShapeDtypeStruct((B, H), jnp.float32),
    mesh=plsc.VectorSubcoreMesh(core_axis_name="c", subcore_axis_name="s",
                                num_cores=NC, num_subcores=NS),
    scratch_types=[
        pltpu.VMEM((NCHUNK, CHUNK), jnp.int32),
        pltpu.VMEM((NCHUNK, CHUNK), jnp.int32),
        pltpu.VMEM((NCHUNK, CHUNK), jnp.int32),
        pltpu.VMEM((BPW, H), jnp.float32),
        pltpu.VMEM((2, CHUNK, H), jnp.float32),
        pltpu.SemaphoreType.DMA,
        pltpu.SemaphoreType.DMA,
        pltpu.SemaphoreType.DMA,
        pltpu.SemaphoreType.DMA,
    ],
)


def _mlp_body(h_ref, w1_ref, b1_ref, w2_ref, b2_ref, out_ref):
    h = h_ref[...]
    a = jnp.dot(h, w1_ref[...], preferred_element_type=jnp.float32) + b1_ref[...]
    a = a * jax.nn.sigmoid(a)
    out_ref[...] = (jnp.dot(a, w2_ref[...], preferred_element_type=jnp.float32)
                    + b2_ref[...])


def _mlp(hidden, W1, b1, W2, b2):
    grid = (B // BM,)
    row_spec = pl.BlockSpec((BM, H), lambda i: (i, 0))
    full = pl.BlockSpec((H, H), lambda i: (0, 0))
    bias = pl.BlockSpec((1, H), lambda i: (0, 0))
    return pl.pallas_call(
        _mlp_body,
        grid=grid,
        in_specs=[row_spec, full, bias, full, bias],
        out_specs=row_spec,
        out_shape=jax.ShapeDtypeStruct((B, H), jnp.float32),
    )(hidden, W1, b1.reshape(1, H), W2, b2.reshape(1, H))


def kernel(perturbation, cell_type, batch, perturb_table, cell_table,
           batch_table, W1, b1, W2, b2):
    ip = perturbation.astype(jnp.int32).reshape(B // CHUNK, CHUNK)
    ic = cell_type.astype(jnp.int32).reshape(B // CHUNK, CHUNK)
    ib = batch.astype(jnp.int32).reshape(B // CHUNK, CHUNK)
    hidden = _gather(perturb_table, cell_table, batch_table, ip, ic, ib)
    return _mlp(hidden, W1, b1, W2, b2)
